# baseline (speedup 1.0000x reference)
import jax
import jax.numpy as jnp
from jax import lax
from jax.experimental import pallas as pl
from jax.experimental.pallas import tpu as pltpu

_SMOOTH = 0.1
_IGNORE = 1
_N = 4096
_V = 32000
_R = 512
_VB = 6400


def _loss_body(tgt_ref, pred_ref, out_ref, m_ref, se_ref, s_ref, tv_ref):
    i = pl.program_id(0)
    j = pl.program_id(1)
    nj = pl.num_programs(1)

    @pl.when(j == 0)
    def _init_row():
        m_ref[...] = jnp.full_like(m_ref, -jnp.inf)
        se_ref[...] = jnp.zeros_like(se_ref)
        s_ref[...] = jnp.zeros_like(s_ref)
        tv_ref[...] = jnp.zeros_like(tv_ref)

    @pl.when(jnp.logical_and(i == 0, j == 0))
    def _init_out():
        out_ref[...] = jnp.zeros_like(out_ref)

    x = pred_ref[...]
    t = tgt_ref[0, :, :]
    bm = jnp.max(x, axis=1, keepdims=True)
    m_old = m_ref[...]
    m_new = jnp.maximum(m_old, bm)
    alpha = jnp.exp(m_old - m_new)
    e = jnp.exp(x - m_new)
    se_ref[...] = se_ref[...] * alpha + jnp.sum(e, axis=1, keepdims=True)
    s_ref[...] += jnp.sum(x, axis=1, keepdims=True)
    col = lax.broadcasted_iota(jnp.int32, x.shape, 1) + j * _VB
    tv_ref[...] += jnp.sum(jnp.where(col == t, x, 0.0), axis=1, keepdims=True)
    m_ref[...] = m_new

    @pl.when(j == nj - 1)
    def _finish_row():
        lse = m_ref[...] + jnp.log(se_ref[...])
        sum_logp = s_ref[...] - _V * lse
        logp_t = tv_ref[...] - lse
        eps = _SMOOTH / (_V - 1)
        coef = 1.0 - _SMOOTH - eps
        loss = jnp.where(t != _IGNORE, -(eps * sum_logp + coef * logp_t), 0.0)
        out_ref[...] += jnp.sum(loss).reshape(1, 1)


def kernel(pred, target):
    tgt3 = target.astype(jnp.int32).reshape(_N // _R, _R, 1)
    out = pl.pallas_call(
        _loss_body,
        grid=(_N // _R, _V // _VB),
        in_specs=[
            pl.BlockSpec((1, _R, 1), lambda i, j: (i, 0, 0)),
            pl.BlockSpec((_R, _VB), lambda i, j: (i, j)),
        ],
        out_specs=pl.BlockSpec((1, 1), lambda i, j: (0, 0)),
        out_shape=jax.ShapeDtypeStruct((1, 1), jnp.float32),
        scratch_shapes=[
            pltpu.VMEM((_R, 1), jnp.float32),
            pltpu.VMEM((_R, 1), jnp.float32),
            pltpu.VMEM((_R, 1), jnp.float32),
            pltpu.VMEM((_R, 1), jnp.float32),
        ],
    )(tgt3, pred)
    return out[0, 0] / _N


# speedup vs baseline: 1.0740x; 1.0740x over previous
import jax
import jax.numpy as jnp
from jax import lax
from jax.experimental import pallas as pl
from jax.experimental.pallas import tpu as pltpu

_SMOOTH = 0.1
_IGNORE = 1
_N = 4096
_V = 32000
_R = 256
_VB = 16000


def _loss_body(tgt_ref, pred_ref, out_ref, m_ref, se_ref, s_ref, tv_ref):
    i = pl.program_id(0)
    j = pl.program_id(1)
    nj = pl.num_programs(1)

    @pl.when(j == 0)
    def _init_row():
        m_ref[...] = jnp.full_like(m_ref, -jnp.inf)
        se_ref[...] = jnp.zeros_like(se_ref)
        s_ref[...] = jnp.zeros_like(s_ref)
        tv_ref[...] = jnp.zeros_like(tv_ref)

    @pl.when(jnp.logical_and(i == 0, j == 0))
    def _init_out():
        out_ref[...] = jnp.zeros_like(out_ref)

    x = pred_ref[...]
    t = tgt_ref[0, :, :]
    bm = jnp.max(x, axis=1, keepdims=True)
    m_old = m_ref[...]
    m_new = jnp.maximum(m_old, bm)
    alpha = jnp.exp(m_old - m_new)
    e = jnp.exp(x - m_new)
    se_ref[...] = se_ref[...] * alpha + jnp.sum(e, axis=1, keepdims=True)
    s_ref[...] += jnp.sum(x, axis=1, keepdims=True)
    col = lax.broadcasted_iota(jnp.int32, x.shape, 1) + j * _VB
    tv_ref[...] += jnp.sum(jnp.where(col == t, x, 0.0), axis=1, keepdims=True)
    m_ref[...] = m_new

    @pl.when(j == nj - 1)
    def _finish_row():
        lse = m_ref[...] + jnp.log(se_ref[...])
        sum_logp = s_ref[...] - _V * lse
        logp_t = tv_ref[...] - lse
        eps = _SMOOTH / (_V - 1)
        coef = 1.0 - _SMOOTH - eps
        loss = jnp.where(t != _IGNORE, -(eps * sum_logp + coef * logp_t), 0.0)
        out_ref[...] += jnp.sum(loss).reshape(1, 1)


def kernel(pred, target):
    tgt3 = target.astype(jnp.int32).reshape(_N // _R, _R, 1)
    out = pl.pallas_call(
        _loss_body,
        grid=(_N // _R, _V // _VB),
        in_specs=[
            pl.BlockSpec((1, _R, 1), lambda i, j: (i, 0, 0)),
            pl.BlockSpec((_R, _VB), lambda i, j: (i, j)),
        ],
        out_specs=pl.BlockSpec((1, 1), lambda i, j: (0, 0)),
        out_shape=jax.ShapeDtypeStruct((1, 1), jnp.float32),
        scratch_shapes=[
            pltpu.VMEM((_R, 1), jnp.float32),
            pltpu.VMEM((_R, 1), jnp.float32),
            pltpu.VMEM((_R, 1), jnp.float32),
            pltpu.VMEM((_R, 1), jnp.float32),
        ],
    )(tgt3, pred)
    return out[0, 0] / _N
